# baseline (speedup 1.0000x reference)
import functools

import jax
import jax.numpy as jnp
from jax import lax
from jax.experimental import pallas as pl
from jax.experimental.pallas import tpu as pltpu
from jax.experimental.pallas import tpu_sc as plsc

NC = 2
NS = 16
NW = NC * NS
CHUNK = 128
N_PAD = 10240
ROWS_PER_TILE = N_PAD // NS
ZCOPIES = ROWS_PER_TILE // CHUNK
DEG_W = 16


def _mesh():
    return plsc.VectorSubcoreMesh(core_axis_name="c", subcore_axis_name="s")


def _deg_body(dst_hbm, out_hbm, idx_v, ones_v, zb_v, acc):
    c = lax.axis_index("c")
    s = lax.axis_index("s")
    wid = c * NS + s
    n_chunks = dst_hbm.shape[1]

    def fill(i, carry):
        ones_v[i, :] = jnp.ones((DEG_W,), jnp.float32)
        zb_v[i, :] = jnp.zeros((DEG_W,), jnp.float32)
        return carry

    lax.fori_loop(0, CHUNK, fill, 0)
    for k in range(ZCOPIES):
        pltpu.sync_copy(zb_v, acc.at[pl.ds(s * ROWS_PER_TILE + k * CHUNK, CHUNK)])
    plsc.subcore_barrier()

    pltpu.sync_copy(dst_hbm.at[wid], idx_v)

    def body(j, carry):
        pltpu.sync_copy(ones_v, acc.at[idx_v.at[j]], add=True)
        return carry

    lax.fori_loop(0, n_chunks, body, 0)
    plsc.subcore_barrier()
    for k in range(ZCOPIES):
        sl = pl.ds(s * ROWS_PER_TILE + k * CHUNK, CHUNK)
        pltpu.sync_copy(acc.at[sl], out_hbm.at[c, sl])


def _deg_call(dst_r):
    n_chunks = dst_r.shape[1]
    return pl.kernel(
        _deg_body,
        out_type=jax.ShapeDtypeStruct((NC, N_PAD, DEG_W), jnp.float32),
        mesh=_mesh(),
        scratch_types=[
            pltpu.VMEM((n_chunks, CHUNK), jnp.int32),
            pltpu.VMEM((CHUNK, DEG_W), jnp.float32),
            pltpu.VMEM((CHUNK, DEG_W), jnp.float32),
            pltpu.VMEM_SHARED((N_PAD, DEG_W), jnp.float32),
        ],
        compiler_params=pltpu.CompilerParams(use_tc_tiling_on_sc=False),
    )(dst_r)


NBUF = 2


def _edge_body(table_hbm, src_hbm, dst_hbm, out_hbm,
               sidx_v, didx_v, rows_v, zb_v, sem, tbl, acc):
    c = lax.axis_index("c")
    s = lax.axis_index("s")
    wid = c * NS + s
    n_chunks = src_hbm.shape[1]
    d = table_hbm.shape[1]

    def fill(i, carry):
        for j in range(d // 16):
            zb_v[i, pl.ds(j * 16, 16)] = jnp.zeros((16,), jnp.float32)
        return carry

    lax.fori_loop(0, CHUNK, fill, 0)
    for k in range(ZCOPIES):
        pltpu.sync_copy(zb_v, acc.at[pl.ds(s * ROWS_PER_TILE + k * CHUNK, CHUNK)])
    pltpu.sync_copy(table_hbm.at[pl.ds(s * ROWS_PER_TILE, ROWS_PER_TILE)],
                    tbl.at[pl.ds(s * ROWS_PER_TILE, ROWS_PER_TILE)])
    plsc.subcore_barrier()

    pltpu.sync_copy(src_hbm.at[wid], sidx_v)
    pltpu.sync_copy(dst_hbm.at[wid], didx_v)

    def body(j, carry):
        pltpu.async_copy(tbl.at[sidx_v.at[j]], rows_v, sem).wait()
        pltpu.sync_copy(rows_v, acc.at[didx_v.at[j]], add=True)
        return carry

    lax.fori_loop(0, n_chunks, body, 0)
    plsc.subcore_barrier()
    for k in range(ZCOPIES):
        sl = pl.ds(s * ROWS_PER_TILE + k * CHUNK, CHUNK)
        pltpu.sync_copy(acc.at[sl], out_hbm.at[c, sl])


def _edge_call(table, src_r, dst_r):
    n_chunks = src_r.shape[1]
    d = table.shape[1]
    return pl.kernel(
        _edge_body,
        out_type=jax.ShapeDtypeStruct((NC, N_PAD, d), jnp.float32),
        mesh=_mesh(),
        scratch_types=[
            pltpu.VMEM((n_chunks, CHUNK), jnp.int32),
            pltpu.VMEM((n_chunks, CHUNK), jnp.int32),
            pltpu.VMEM((CHUNK, d), jnp.float32),
            pltpu.VMEM((CHUNK, d), jnp.float32),
            pltpu.SemaphoreType.DMA,
            pltpu.VMEM_SHARED((N_PAD, d), jnp.float32),
            pltpu.VMEM_SHARED((N_PAD, d), jnp.float32),
        ],
        compiler_params=pltpu.CompilerParams(use_tc_tiling_on_sc=False),
    )(table, src_r, dst_r)


BLK = 640


def _dis(deg_ref):
    s = deg_ref[0] + deg_ref[1]
    return lax.rsqrt(s[:, 0:1] + 1.0)


_DEG_SPEC = pl.BlockSpec((NC, BLK, DEG_W), lambda i: (0, i, 0))


def _mm_scale_body(x_ref, w_ref, deg_ref, o_ref):
    h = jnp.dot(x_ref[...], w_ref[...], preferred_element_type=jnp.float32)
    o_ref[...] = h * _dis(deg_ref)


def _mm_scale_call(x, w, degp):
    d_in = x.shape[1]
    d_out = w.shape[1]
    return pl.pallas_call(
        _mm_scale_body,
        grid=(N_PAD // BLK,),
        in_specs=[
            pl.BlockSpec((BLK, d_in), lambda i: (i, 0)),
            pl.BlockSpec((d_in, d_out), lambda i: (0, 0)),
            _DEG_SPEC,
        ],
        out_specs=pl.BlockSpec((BLK, d_out), lambda i: (i, 0)),
        out_shape=jax.ShapeDtypeStruct((N_PAD, d_out), jnp.float32),
    )(x, w, degp)


def _layer2_body(p0_ref, p1_ref, g1_ref, deg_ref, b_ref, w_ref, o_ref):
    dis = _dis(deg_ref)
    t = dis * (p0_ref[...] + p1_ref[...] + g1_ref[...]) + b_ref[...]
    t = jnp.where(t >= 0, t, 0.01 * t)
    h2 = jnp.dot(t, w_ref[...], preferred_element_type=jnp.float32)
    o_ref[...] = h2 * dis


def _layer2_call(p0, p1, g1, degp, b, w):
    d = g1.shape[1]
    return pl.pallas_call(
        _layer2_body,
        grid=(N_PAD // BLK,),
        in_specs=[
            pl.BlockSpec((BLK, d), lambda i: (i, 0)),
            pl.BlockSpec((BLK, d), lambda i: (i, 0)),
            pl.BlockSpec((BLK, d), lambda i: (i, 0)),
            _DEG_SPEC,
            pl.BlockSpec((1, d), lambda i: (0, 0)),
            pl.BlockSpec((d, d), lambda i: (0, 0)),
        ],
        out_specs=pl.BlockSpec((BLK, d), lambda i: (i, 0)),
        out_shape=jax.ShapeDtypeStruct((N_PAD, d), jnp.float32),
    )(p0, p1, g1, degp, b, w)


def _final_body(q0_ref, q1_ref, g2_ref, deg_ref, b_ref, w_ref, fb_ref, o_ref):
    t = _dis(deg_ref) * (q0_ref[...] + q1_ref[...] + g2_ref[...]) + b_ref[...]
    o_ref[...] = (
        jnp.dot(t, w_ref[...], preferred_element_type=jnp.float32) + fb_ref[...]
    )


def _final_call(q0, q1, g2, degp, b, w, fb):
    d = g2.shape[1]
    d_out = w.shape[1]
    return pl.pallas_call(
        _final_body,
        grid=(N_PAD // BLK,),
        in_specs=[
            pl.BlockSpec((BLK, d), lambda i: (i, 0)),
            pl.BlockSpec((BLK, d), lambda i: (i, 0)),
            pl.BlockSpec((BLK, d), lambda i: (i, 0)),
            _DEG_SPEC,
            pl.BlockSpec((1, d), lambda i: (0, 0)),
            pl.BlockSpec((d, d_out), lambda i: (0, 0)),
            pl.BlockSpec((1, d_out), lambda i: (0, 0)),
        ],
        out_specs=pl.BlockSpec((BLK, d_out), lambda i: (i, 0)),
        out_shape=jax.ShapeDtypeStruct((N_PAD, d_out), jnp.float32),
    )(q0, q1, g2, degp, b, w, fb)


def kernel(x, edge_index, W1, b1, W2, b2, fc_W, fc_b):
    n = x.shape[0]
    e = edge_index.shape[1]
    src = edge_index[0]
    dst = edge_index[1]

    per_xfer = NW * CHUNK
    n_chunks = -(-e // per_xfer)
    n_chunks = -(-n_chunks // NBUF) * NBUF
    e_pad = per_xfer * n_chunks
    if e_pad != e:
        pad = e_pad - e
        src = jnp.concatenate([src, jnp.zeros((pad,), jnp.int32)])
        dst = jnp.concatenate([dst, jnp.full((pad,), n, jnp.int32)])
    src_r = src.reshape(NW, n_chunks, CHUNK)
    dst_r = dst.reshape(NW, n_chunks, CHUNK)

    degp = _deg_call(dst_r)
    g1 = _mm_scale_call(x, W1, degp)
    p = _edge_call(g1, src_r, dst_r)
    g2 = _layer2_call(p[0], p[1], g1, degp, b1[None, :], W2)
    q = _edge_call(g2, src_r, dst_r)
    outp = _final_call(q[0], q[1], g2, degp, b2[None, :], fc_W, fc_b[None, :])
    return outp[:n]


# speedup vs baseline: 1.4012x; 1.4012x over previous
import functools

import jax
import jax.numpy as jnp
from jax import lax
from jax.experimental import pallas as pl
from jax.experimental.pallas import tpu as pltpu
from jax.experimental.pallas import tpu_sc as plsc

NC = 2
NS = 16
NW = NC * NS
CHUNK = 128
N_PAD = 10240
ROWS_PER_TILE = N_PAD // NS
ZCOPIES = ROWS_PER_TILE // CHUNK
DEG_W = 16


def _mesh():
    return plsc.VectorSubcoreMesh(core_axis_name="c", subcore_axis_name="s")


def _deg_body(dst_hbm, out_hbm, idx_v, ones_v, zb_v, acc):
    c = lax.axis_index("c")
    s = lax.axis_index("s")
    wid = c * NS + s
    n_chunks = dst_hbm.shape[1]

    def fill(i, carry):
        ones_v[i, :] = jnp.ones((DEG_W,), jnp.float32)
        zb_v[i, :] = jnp.zeros((DEG_W,), jnp.float32)
        return carry

    lax.fori_loop(0, CHUNK, fill, 0)
    for k in range(ZCOPIES):
        pltpu.sync_copy(zb_v, acc.at[pl.ds(s * ROWS_PER_TILE + k * CHUNK, CHUNK)])
    plsc.subcore_barrier()

    pltpu.sync_copy(dst_hbm.at[wid], idx_v)

    def body(j, carry):
        pltpu.sync_copy(ones_v, acc.at[idx_v.at[j]], add=True)
        return carry

    lax.fori_loop(0, n_chunks, body, 0)
    plsc.subcore_barrier()
    for k in range(ZCOPIES):
        sl = pl.ds(s * ROWS_PER_TILE + k * CHUNK, CHUNK)
        pltpu.sync_copy(acc.at[sl], out_hbm.at[c, sl])


def _deg_call(dst_r):
    n_chunks = dst_r.shape[1]
    return pl.kernel(
        _deg_body,
        out_type=jax.ShapeDtypeStruct((NC, N_PAD, DEG_W), jnp.float32),
        mesh=_mesh(),
        scratch_types=[
            pltpu.VMEM((n_chunks, CHUNK), jnp.int32),
            pltpu.VMEM((CHUNK, DEG_W), jnp.float32),
            pltpu.VMEM((CHUNK, DEG_W), jnp.float32),
            pltpu.VMEM_SHARED((N_PAD, DEG_W), jnp.float32),
        ],
        compiler_params=pltpu.CompilerParams(use_tc_tiling_on_sc=False),
    )(dst_r)


NBUF = 2


def _edge_body(table_hbm, src_hbm, dst_hbm, zrow_hbm, out_hbm,
               sidx_v, didx_v, rows_v, sem, tbl, acc):
    c = lax.axis_index("c")
    s = lax.axis_index("s")
    wid = c * NS + s
    n_chunks = src_hbm.shape[1]

    tile_sl = pl.ds(s * ROWS_PER_TILE, ROWS_PER_TILE)
    pltpu.sync_copy(zrow_hbm, acc.at[tile_sl])
    pltpu.sync_copy(table_hbm.at[tile_sl], tbl.at[tile_sl])
    plsc.subcore_barrier()

    pltpu.sync_copy(src_hbm.at[wid], sidx_v)
    pltpu.sync_copy(dst_hbm.at[wid], didx_v)

    def body(j, carry):
        pltpu.async_copy(tbl.at[sidx_v.at[j]], rows_v, sem).wait()
        pltpu.sync_copy(rows_v, acc.at[didx_v.at[j]], add=True)
        return carry

    lax.fori_loop(0, n_chunks, body, 0)
    plsc.subcore_barrier()
    pltpu.sync_copy(acc.at[tile_sl], out_hbm.at[c, tile_sl])


def _edge_call(table, src_r, dst_r, zrow):
    n_chunks = src_r.shape[1]
    d = table.shape[1]
    return pl.kernel(
        _edge_body,
        out_type=jax.ShapeDtypeStruct((NC, N_PAD, d), jnp.bfloat16),
        mesh=_mesh(),
        scratch_types=[
            pltpu.VMEM((n_chunks, CHUNK), jnp.int32),
            pltpu.VMEM((n_chunks, CHUNK), jnp.int32),
            pltpu.VMEM((CHUNK, d), jnp.bfloat16),
            pltpu.SemaphoreType.DMA,
            pltpu.VMEM_SHARED((N_PAD, d), jnp.bfloat16),
            pltpu.VMEM_SHARED((N_PAD, d), jnp.bfloat16),
        ],
        compiler_params=pltpu.CompilerParams(use_tc_tiling_on_sc=False),
    )(table, src_r, dst_r, zrow)


BLK = 640


def _dis(deg_ref):
    s = deg_ref[0] + deg_ref[1]
    return lax.rsqrt(s[:, 0:1] + 1.0)


_DEG_SPEC = pl.BlockSpec((NC, BLK, DEG_W), lambda i: (0, i, 0))


def _mm_scale_body(x_ref, w_ref, deg_ref, o_ref):
    h = jnp.dot(x_ref[...], w_ref[...], preferred_element_type=jnp.float32)
    o_ref[...] = (h * _dis(deg_ref)).astype(jnp.bfloat16)


def _mm_scale_call(x, w, degp):
    d_in = x.shape[1]
    d_out = w.shape[1]
    return pl.pallas_call(
        _mm_scale_body,
        grid=(N_PAD // BLK,),
        in_specs=[
            pl.BlockSpec((BLK, d_in), lambda i: (i, 0)),
            pl.BlockSpec((d_in, d_out), lambda i: (0, 0)),
            _DEG_SPEC,
        ],
        out_specs=pl.BlockSpec((BLK, d_out), lambda i: (i, 0)),
        out_shape=jax.ShapeDtypeStruct((N_PAD, d_out), jnp.bfloat16),
    )(x, w, degp)


def _layer2_body(p0_ref, p1_ref, g1_ref, deg_ref, b_ref, w_ref, o_ref):
    dis = _dis(deg_ref)
    agg = (p0_ref[...].astype(jnp.float32) + p1_ref[...].astype(jnp.float32)
           + g1_ref[...].astype(jnp.float32))
    t = dis * agg + b_ref[...]
    t = jnp.where(t >= 0, t, 0.01 * t)
    h2 = jnp.dot(t, w_ref[...], preferred_element_type=jnp.float32)
    o_ref[...] = (h2 * dis).astype(jnp.bfloat16)


def _layer2_call(p0, p1, g1, degp, b, w):
    d = g1.shape[1]
    return pl.pallas_call(
        _layer2_body,
        grid=(N_PAD // BLK,),
        in_specs=[
            pl.BlockSpec((BLK, d), lambda i: (i, 0)),
            pl.BlockSpec((BLK, d), lambda i: (i, 0)),
            pl.BlockSpec((BLK, d), lambda i: (i, 0)),
            _DEG_SPEC,
            pl.BlockSpec((1, d), lambda i: (0, 0)),
            pl.BlockSpec((d, d), lambda i: (0, 0)),
        ],
        out_specs=pl.BlockSpec((BLK, d), lambda i: (i, 0)),
        out_shape=jax.ShapeDtypeStruct((N_PAD, d), jnp.bfloat16),
    )(p0, p1, g1, degp, b, w)


def _final_body(q0_ref, q1_ref, g2_ref, deg_ref, b_ref, w_ref, fb_ref, o_ref):
    agg = (q0_ref[...].astype(jnp.float32) + q1_ref[...].astype(jnp.float32)
           + g2_ref[...].astype(jnp.float32))
    t = _dis(deg_ref) * agg + b_ref[...]
    o_ref[...] = (
        jnp.dot(t, w_ref[...], preferred_element_type=jnp.float32) + fb_ref[...]
    )


def _final_call(q0, q1, g2, degp, b, w, fb):
    d = g2.shape[1]
    d_out = w.shape[1]
    return pl.pallas_call(
        _final_body,
        grid=(N_PAD // BLK,),
        in_specs=[
            pl.BlockSpec((BLK, d), lambda i: (i, 0)),
            pl.BlockSpec((BLK, d), lambda i: (i, 0)),
            pl.BlockSpec((BLK, d), lambda i: (i, 0)),
            _DEG_SPEC,
            pl.BlockSpec((1, d), lambda i: (0, 0)),
            pl.BlockSpec((d, d_out), lambda i: (0, 0)),
            pl.BlockSpec((1, d_out), lambda i: (0, 0)),
        ],
        out_specs=pl.BlockSpec((BLK, d_out), lambda i: (i, 0)),
        out_shape=jax.ShapeDtypeStruct((N_PAD, d_out), jnp.float32),
    )(q0, q1, g2, degp, b, w, fb)


def kernel(x, edge_index, W1, b1, W2, b2, fc_W, fc_b):
    n = x.shape[0]
    e = edge_index.shape[1]
    src = edge_index[0]
    dst = edge_index[1]

    per_xfer = NW * CHUNK
    n_chunks = -(-e // per_xfer)
    n_chunks = -(-n_chunks // NBUF) * NBUF
    e_pad = per_xfer * n_chunks
    if e_pad != e:
        pad = e_pad - e
        src = jnp.concatenate([src, jnp.zeros((pad,), jnp.int32)])
        dst = jnp.concatenate([dst, jnp.full((pad,), n, jnp.int32)])
    src_r = src.reshape(NW, n_chunks, CHUNK)
    dst_r = dst.reshape(NW, n_chunks, CHUNK)

    zrow = jnp.zeros((ROWS_PER_TILE, W1.shape[1]), jnp.bfloat16)
    degp = _deg_call(dst_r)
    g1 = _mm_scale_call(x, W1, degp)
    p = _edge_call(g1, src_r, dst_r, zrow)
    g2 = _layer2_call(p[0], p[1], g1, degp, b1[None, :], W2)
    q = _edge_call(g2, src_r, dst_r, zrow)
    outp = _final_call(q[0], q[1], g2, degp, b2[None, :], fc_W, fc_b[None, :])
    return outp[:n]
